# Initial kernel scaffold; baseline (speedup 1.0000x reference)
import jax
import jax.numpy as jnp
from jax.experimental import pallas as pl


def kernel(embeddings, token_type_ids, attention_mask, W, b):
    raise NotImplementedError("write your pallas kernel here")


# speedup vs baseline: 24.3557x; 24.3557x over previous
import functools

import numpy as np
import jax
import jax.numpy as jnp
from jax import lax
from jax.experimental import pallas as pl
from jax.experimental.pallas import tpu as pltpu
from jax.experimental.pallas import tpu_sc as plsc

L = 2048
A = 16
B = 2
D = 2048
P = L * A - A * (A - 1) // 2
PP = 32768
LP = L + 16
NSUB = 16
CHUNK = PP // NSUB
NEG = -1e30


def _build_pair_indices():
    i = np.arange(L).reshape(-1, 1)
    j = np.arange(L).reshape(1, -1)
    cond = (j >= i) & (j <= i + A - 1)
    idx = np.argwhere(cond)
    si = np.full((PP,), L, dtype=np.int32)
    ei = np.full((PP,), L, dtype=np.int32)
    si[:P] = idx[:, 0]
    ei[:P] = idx[:, 1]
    return jnp.asarray(si), jnp.asarray(ei)


_SI, _EI = _build_pair_indices()


BLK = 512


def _matvec_body(emb_ref, tt_ref, am_ref, w2_ref, b_ref, s_ref, e_ref, m_ref):
    blk = emb_ref[0]
    se = jnp.dot(blk, w2_ref[...], preferred_element_type=jnp.float32,
                 precision=jax.lax.Precision.HIGHEST)
    bias = b_ref[0, 0]
    s_ref[0, 0, :] = se[:, 0] + bias
    e_ref[0, 0, :] = se[:, 1]
    m_ref[0, 0, :] = (tt_ref[0, 0, :] * am_ref[0, 0, :]).astype(jnp.float32)


def _matvec(emb, tt, am, w2, b2):
    vec = pl.BlockSpec((1, 1, BLK), lambda i, l: (i, 0, l))
    s, e, m = pl.pallas_call(
        _matvec_body,
        grid=(B, L // BLK),
        in_specs=[
            pl.BlockSpec((1, BLK, D), lambda i, l: (i, l, 0)),
            vec,
            vec,
            pl.BlockSpec((D, 2), lambda i, l: (0, 0)),
            pl.BlockSpec((1, 1), lambda i, l: (0, 0)),
        ],
        out_specs=[vec, vec, vec],
        out_shape=[
            jax.ShapeDtypeStruct((B, 1, L), jnp.float32),
            jax.ShapeDtypeStruct((B, 1, L), jnp.float32),
            jax.ShapeDtypeStruct((B, 1, L), jnp.float32),
        ],
    )(emb, tt.reshape(B, 1, L), am.reshape(B, 1, L), w2, b2)
    return s.reshape(B, L), e.reshape(B, L), m.reshape(B, L)


def _sc_band_softmax(sx, ex, mx, si, ei):
    mesh = plsc.VectorSubcoreMesh(core_axis_name="c", subcore_axis_name="s")

    @functools.partial(
        pl.kernel,
        mesh=mesh,
        out_type=jax.ShapeDtypeStruct((B, PP), jnp.float32),
        compiler_params=pltpu.CompilerParams(needs_layout_passes=False),
        scratch_types=[
            pltpu.VMEM((LP,), jnp.float32),
            pltpu.VMEM((LP,), jnp.float32),
            pltpu.VMEM((LP,), jnp.float32),
            pltpu.VMEM((CHUNK,), jnp.int32),
            pltpu.VMEM((CHUNK,), jnp.int32),
            pltpu.VMEM((CHUNK,), jnp.float32),
            pltpu.VMEM((16,), jnp.float32),
            pltpu.VMEM((NSUB * 16,), jnp.float32),
            pltpu.VMEM_SHARED((NSUB * 16,), jnp.float32),
            pltpu.VMEM_SHARED((NSUB * 16,), jnp.float32),
        ],
    )
    def k(sx_h, ex_h, mx_h, si_h, ei_h, out_h,
          s_v, e_v, m_v, si_v, ei_v, lg_v, st_v, rd_v, shr_max, shr_sum):
        c = lax.axis_index("c")
        sid = lax.axis_index("s")
        base = sid * CHUNK

        pltpu.sync_copy(sx_h.at[c], s_v)
        pltpu.sync_copy(ex_h.at[c], e_v)
        pltpu.sync_copy(mx_h.at[c], m_v)
        pltpu.sync_copy(si_h.at[pl.ds(base, CHUNK)], si_v)
        pltpu.sync_copy(ei_h.at[pl.ds(base, CHUNK)], ei_v)

        def body_logits(kk, vmax):
            sl = pl.ds(kk * 16, 16)
            iv = si_v[sl]
            jv = ei_v[sl]
            sv = plsc.load_gather(s_v, [iv])
            ev = plsc.load_gather(e_v, [jv])
            mi = plsc.load_gather(m_v, [iv])
            mj = plsc.load_gather(m_v, [jv])
            lg = sv + ev - 1e7 * (1.0 - mi * mj)
            lg_v[sl] = lg
            return jnp.maximum(vmax, lg)

        vmax = lax.fori_loop(0, CHUNK // 16, body_logits,
                             jnp.full((16,), NEG, jnp.float32))

        st_v[...] = vmax
        pltpu.sync_copy(st_v, shr_max.at[pl.ds(sid * 16, 16)])
        plsc.subcore_barrier()
        pltpu.sync_copy(shr_max, rd_v)
        gv = jnp.full((16,), NEG, jnp.float32)
        for i in range(NSUB):
            gv = jnp.maximum(gv, rd_v[pl.ds(i * 16, 16)])
        gmax = lax.broadcast(jnp.max(gv), (16,))

        def body_exp(kk, vsum):
            sl = pl.ds(kk * 16, 16)
            pv = jnp.exp(lg_v[sl] - gmax)
            lg_v[sl] = pv
            return vsum + pv

        vsum = lax.fori_loop(0, CHUNK // 16, body_exp,
                             jnp.zeros((16,), jnp.float32))

        st_v[...] = vsum
        pltpu.sync_copy(st_v, shr_sum.at[pl.ds(sid * 16, 16)])
        plsc.subcore_barrier()
        pltpu.sync_copy(shr_sum, rd_v)
        tv = jnp.zeros((16,), jnp.float32)
        for i in range(NSUB):
            tv = tv + rd_v[pl.ds(i * 16, 16)]
        den = lax.broadcast(jnp.sum(tv), (16,))
        inv = 1.0 / den
        inv = inv * (2.0 - den * inv)
        inv = inv * (2.0 - den * inv)

        def body_norm(kk, carry):
            sl = pl.ds(kk * 16, 16)
            lg_v[sl] = lg_v[sl] * inv
            return carry

        lax.fori_loop(0, CHUNK // 16, body_norm, 0)
        pltpu.sync_copy(lg_v, out_h.at[c, pl.ds(base, CHUNK)])

    return k(sx, ex, mx, si, ei)


def kernel(embeddings, token_type_ids, attention_mask, W, b):
    w2 = jnp.zeros((D, 2), jnp.float32)
    w2 = w2.at[: D // 2, 0].set(W[:, 0]).at[D // 2 :, 1].set(W[:, 0])
    b2 = b.reshape(1, 1)

    s, e, m = _matvec(embeddings, token_type_ids, attention_mask, w2, b2)

    pad_neg = jnp.full((B, LP - L), NEG, jnp.float32)
    pad_zero = jnp.zeros((B, LP - L), jnp.float32)
    sx = jnp.concatenate([s, pad_neg], axis=1)
    ex = jnp.concatenate([e, pad_zero], axis=1)
    mx = jnp.concatenate([m, pad_zero], axis=1)

    out = _sc_band_softmax(sx, ex, mx, _SI, _EI)
    return out[:, :P]


# speedup vs baseline: 25.9515x; 1.0655x over previous
import functools

import numpy as np
import jax
import jax.numpy as jnp
from jax import lax
from jax.experimental import pallas as pl
from jax.experimental.pallas import tpu as pltpu
from jax.experimental.pallas import tpu_sc as plsc

L = 2048
A = 16
B = 2
D = 2048
P = L * A - A * (A - 1) // 2
PP = 32768
LP = L + 16
NSUB = 16
CHUNK = PP // NSUB
ROWS = L // NSUB
TAIL = P - (NSUB - 1) * CHUNK
NEG = -1e30


def _build_pair_indices():
    i = np.arange(L).reshape(-1, 1)
    j = np.arange(L).reshape(1, -1)
    cond = (j >= i) & (j <= i + A - 1)
    idx = np.argwhere(cond)
    si = np.full((CHUNK,), L, dtype=np.int32)
    ei = np.full((CHUNK,), L, dtype=np.int32)
    base = (NSUB - 1) * CHUNK
    si[:TAIL] = idx[base:, 0]
    ei[:TAIL] = idx[base:, 1]
    return jnp.asarray(si), jnp.asarray(ei)


_SI_TAIL, _EI_TAIL = _build_pair_indices()


BLK = 512


def _matvec_body(emb_ref, tt_ref, am_ref, w2_ref, b_ref, s_ref, e_ref, m_ref):
    blk = emb_ref[0]
    se = jnp.dot(blk, w2_ref[...], preferred_element_type=jnp.float32,
                 precision=jax.lax.Precision.HIGHEST)
    bias = b_ref[0, 0]
    s_ref[0, 0, :] = se[:, 0] + bias
    e_ref[0, 0, :] = se[:, 1]
    m_ref[0, 0, :] = (tt_ref[0, 0, :] * am_ref[0, 0, :]).astype(jnp.float32)


def _matvec(emb, tt, am, w2, b2):
    vec = pl.BlockSpec((1, 1, BLK), lambda i, l: (i, 0, l))
    s, e, m = pl.pallas_call(
        _matvec_body,
        grid=(B, L // BLK),
        in_specs=[
            pl.BlockSpec((1, BLK, D), lambda i, l: (i, l, 0)),
            vec,
            vec,
            pl.BlockSpec((D, 2), lambda i, l: (0, 0)),
            pl.BlockSpec((1, 1), lambda i, l: (0, 0)),
        ],
        out_specs=[vec, vec, vec],
        out_shape=[
            jax.ShapeDtypeStruct((B, 1, L), jnp.float32),
            jax.ShapeDtypeStruct((B, 1, L), jnp.float32),
            jax.ShapeDtypeStruct((B, 1, L), jnp.float32),
        ],
    )(emb, tt.reshape(B, 1, L), am.reshape(B, 1, L), w2, b2)
    return s.reshape(B, L), e.reshape(B, L), m.reshape(B, L)


def _sc_band_softmax(s, e, m, si_tail, ei_tail):
    mesh = plsc.VectorSubcoreMesh(core_axis_name="c", subcore_axis_name="s")

    @functools.partial(
        pl.kernel,
        mesh=mesh,
        out_type=jax.ShapeDtypeStruct((B, PP), jnp.float32),
        compiler_params=pltpu.CompilerParams(needs_layout_passes=False),
        scratch_types=[
            pltpu.VMEM((LP,), jnp.float32),
            pltpu.VMEM((LP,), jnp.float32),
            pltpu.VMEM((LP,), jnp.float32),
            pltpu.VMEM((CHUNK,), jnp.int32),
            pltpu.VMEM((CHUNK,), jnp.int32),
            pltpu.VMEM((ROWS + 128,), jnp.float32),
            pltpu.VMEM((ROWS + 128,), jnp.float32),
            pltpu.VMEM((ROWS + 128,), jnp.float32),
            pltpu.VMEM((CHUNK,), jnp.float32),
            pltpu.VMEM((16,), jnp.float32),
            pltpu.VMEM((NSUB * 16,), jnp.float32),
            pltpu.VMEM_SHARED((NSUB * 16,), jnp.float32),
            pltpu.VMEM_SHARED((NSUB * 16,), jnp.float32),
        ],
    )
    def k(s_h, e_h, m_h, si_h, ei_h, out_h,
          s_v, e_v, m_v, si_v, ei_v, e_lv, m_lv, s_lv,
          lg_v, st_v, rd_v, shr_max, shr_sum):
        c = lax.axis_index("c")
        sid = lax.axis_index("s")
        base = sid * CHUNK
        row0 = sid * ROWS
        neg16 = jnp.full((16,), NEG, jnp.float32)

        hoff = c * L + row0

        @pl.when(sid < NSUB - 1)
        def _regular():
            pltpu.sync_copy(e_h.at[pl.ds(hoff, ROWS + 128)], e_lv)
            pltpu.sync_copy(m_h.at[pl.ds(hoff, ROWS + 128)], m_lv)
            pltpu.sync_copy(s_h.at[pl.ds(hoff, ROWS + 128)], s_lv)

            def body(kk, _):
                kv = lax.broadcast(kk, (16,))
                sv = plsc.load_gather(s_lv, [kv])
                miv = plsc.load_gather(m_lv, [kv])
                ev = e_lv[pl.ds(kk, 16)]
                mj = m_lv[pl.ds(kk, 16)]
                lg = sv + ev - 1e7 * (1.0 - miv * mj)
                lg_v[pl.ds(kk * 16, 16)] = lg
                return 0

            lax.fori_loop(0, ROWS, body, 0)

        @pl.when(sid == NSUB - 1)
        def _tail():
            pltpu.sync_copy(s_h.at[pl.ds(c * L, L)], s_v.at[pl.ds(0, L)])
            pltpu.sync_copy(e_h.at[pl.ds(c * L, L)], e_v.at[pl.ds(0, L)])
            pltpu.sync_copy(m_h.at[pl.ds(c * L, L)], m_v.at[pl.ds(0, L)])
            s_v[pl.ds(L, 16)] = neg16
            e_v[pl.ds(L, 16)] = jnp.zeros((16,), jnp.float32)
            m_v[pl.ds(L, 16)] = jnp.zeros((16,), jnp.float32)
            pltpu.sync_copy(si_h, si_v)
            pltpu.sync_copy(ei_h, ei_v)

            def body(kk, _):
                sl = pl.ds(kk * 16, 16)
                iv = si_v[sl]
                jv = ei_v[sl]
                sv = plsc.load_gather(s_v, [iv])
                ev = plsc.load_gather(e_v, [jv])
                mi = plsc.load_gather(m_v, [iv])
                mj = plsc.load_gather(m_v, [jv])
                lg_v[sl] = sv + ev - 1e7 * (1.0 - mi * mj)
                return 0

            lax.fori_loop(0, CHUNK // 16, body, 0)

        def body_max(kk, vmax):
            return jnp.maximum(vmax, lg_v[pl.ds(kk * 16, 16)])

        vmax = lax.fori_loop(0, CHUNK // 16, body_max, neg16)

        st_v[...] = vmax
        pltpu.sync_copy(st_v, shr_max.at[pl.ds(sid * 16, 16)])
        plsc.subcore_barrier()
        pltpu.sync_copy(shr_max, rd_v)
        gv = neg16
        for i in range(NSUB):
            gv = jnp.maximum(gv, rd_v[pl.ds(i * 16, 16)])
        gmax = lax.broadcast(jnp.max(gv), (16,))

        def body_exp(kk, vsum):
            sl = pl.ds(kk * 16, 16)
            pv = jnp.exp(lg_v[sl] - gmax)
            lg_v[sl] = pv
            return vsum + pv

        vsum = lax.fori_loop(0, CHUNK // 16, body_exp,
                             jnp.zeros((16,), jnp.float32))

        st_v[...] = vsum
        pltpu.sync_copy(st_v, shr_sum.at[pl.ds(sid * 16, 16)])
        plsc.subcore_barrier()
        pltpu.sync_copy(shr_sum, rd_v)
        tv = jnp.zeros((16,), jnp.float32)
        for i in range(NSUB):
            tv = tv + rd_v[pl.ds(i * 16, 16)]
        den = lax.broadcast(jnp.sum(tv), (16,))
        inv = 1.0 / den
        inv = inv * (2.0 - den * inv)
        inv = inv * (2.0 - den * inv)

        def body_norm(kk, carry):
            sl = pl.ds(kk * 16, 16)
            lg_v[sl] = lg_v[sl] * inv
            return carry

        lax.fori_loop(0, CHUNK // 16, body_norm, 0)

        pltpu.sync_copy(lg_v, out_h.at[c, pl.ds(base, CHUNK)])

    return k(s, e, m, si_tail, ei_tail)


def kernel(embeddings, token_type_ids, attention_mask, W, b):
    w2 = jnp.zeros((D, 2), jnp.float32)
    w2 = w2.at[: D // 2, 0].set(W[:, 0]).at[D // 2 :, 1].set(W[:, 0])
    b2 = b.reshape(1, 1)

    s, e, m = _matvec(embeddings, token_type_ids, attention_mask, w2, b2)
    out = _sc_band_softmax(s.reshape(-1), e.reshape(-1), m.reshape(-1),
                           _SI_TAIL, _EI_TAIL)
    return out[:, :P]


# speedup vs baseline: 35.7344x; 1.3770x over previous
import functools

import numpy as np
import jax
import jax.numpy as jnp
from jax import lax
from jax.experimental import pallas as pl
from jax.experimental.pallas import tpu as pltpu
from jax.experimental.pallas import tpu_sc as plsc

L = 2048
A = 16
B = 2
D = 2048
P = L * A - A * (A - 1) // 2
PP = 32768
LP = L + 16
NSUB = 16
CHUNK = PP // NSUB
ROWS = L // NSUB
TAIL = P - (NSUB - 1) * CHUNK
NEG = -1e30


def _build_pair_indices():
    i = np.arange(L).reshape(-1, 1)
    j = np.arange(L).reshape(1, -1)
    cond = (j >= i) & (j <= i + A - 1)
    idx = np.argwhere(cond)
    si = np.full((CHUNK,), L, dtype=np.int32)
    ei = np.full((CHUNK,), L, dtype=np.int32)
    base = (NSUB - 1) * CHUNK
    si[:TAIL] = idx[base:, 0]
    ei[:TAIL] = idx[base:, 1]
    return jnp.asarray(si), jnp.asarray(ei)


_SI_TAIL, _EI_TAIL = _build_pair_indices()


BLK = 512


def _matvec_body(emb_ref, tt_ref, am_ref, w2_ref, b_ref, s_ref, e_ref, m_ref):
    blk = emb_ref[0]
    se = jnp.dot(blk, w2_ref[...], preferred_element_type=jnp.float32)
    bias = b_ref[0, 0]
    s_ref[0, 0, :] = se[:, 0] + bias
    e_ref[0, 0, :] = se[:, 1]
    m_ref[0, 0, :] = (tt_ref[0, 0, :] * am_ref[0, 0, :]).astype(jnp.float32)


def _matvec(emb, tt, am, w2, b2):
    vec = pl.BlockSpec((1, 1, BLK), lambda i, l: (i, 0, l))
    s, e, m = pl.pallas_call(
        _matvec_body,
        grid=(B, L // BLK),
        in_specs=[
            pl.BlockSpec((1, BLK, D), lambda i, l: (i, l, 0)),
            vec,
            vec,
            pl.BlockSpec((D, 2), lambda i, l: (0, 0)),
            pl.BlockSpec((1, 1), lambda i, l: (0, 0)),
        ],
        out_specs=[vec, vec, vec],
        out_shape=[
            jax.ShapeDtypeStruct((B, 1, L), jnp.float32),
            jax.ShapeDtypeStruct((B, 1, L), jnp.float32),
            jax.ShapeDtypeStruct((B, 1, L), jnp.float32),
        ],
    )(emb, tt.reshape(B, 1, L), am.reshape(B, 1, L), w2, b2)
    return s.reshape(B, L), e.reshape(B, L), m.reshape(B, L)


def _sc_band_softmax(s, e, m, si_tail, ei_tail):
    mesh = plsc.VectorSubcoreMesh(core_axis_name="c", subcore_axis_name="s")

    @functools.partial(
        pl.kernel,
        mesh=mesh,
        out_type=jax.ShapeDtypeStruct((B, PP), jnp.float32),
        compiler_params=pltpu.CompilerParams(needs_layout_passes=False),
        scratch_types=[
            pltpu.VMEM((LP,), jnp.float32),
            pltpu.VMEM((LP,), jnp.float32),
            pltpu.VMEM((LP,), jnp.float32),
            pltpu.VMEM((CHUNK,), jnp.int32),
            pltpu.VMEM((CHUNK,), jnp.int32),
            pltpu.VMEM((ROWS + 128,), jnp.float32),
            pltpu.VMEM((ROWS + 128,), jnp.float32),
            pltpu.VMEM((ROWS + 128,), jnp.float32),
            pltpu.VMEM((CHUNK,), jnp.float32),
            pltpu.VMEM((16,), jnp.float32),
            pltpu.VMEM((NSUB * 16,), jnp.float32),
            pltpu.VMEM_SHARED((NSUB * 16,), jnp.float32),
            pltpu.VMEM_SHARED((NSUB * 16,), jnp.float32),
        ],
    )
    def k(s_h, e_h, m_h, si_h, ei_h, out_h,
          s_v, e_v, m_v, si_v, ei_v, e_lv, m_lv, s_lv,
          lg_v, st_v, rd_v, shr_max, shr_sum):
        c = lax.axis_index("c")
        sid = lax.axis_index("s")
        base = sid * CHUNK
        row0 = sid * ROWS
        neg16 = jnp.full((16,), NEG, jnp.float32)

        hoff = c * L + row0

        @pl.when(sid < NSUB - 1)
        def _regular():
            pltpu.sync_copy(e_h.at[pl.ds(hoff, ROWS + 128)], e_lv)
            pltpu.sync_copy(m_h.at[pl.ds(hoff, ROWS + 128)], m_lv)
            pltpu.sync_copy(s_h.at[pl.ds(hoff, ROWS + 128)], s_lv)

            def body(kk, _):
                kv = lax.broadcast(kk, (16,))
                sv = plsc.load_gather(s_lv, [kv])
                miv = plsc.load_gather(m_lv, [kv])
                ev = e_lv[pl.ds(kk, 16)]
                mj = m_lv[pl.ds(kk, 16)]
                lg = sv + ev - 1e7 * (1.0 - miv * mj)
                lg_v[pl.ds(kk * 16, 16)] = lg
                return 0

            lax.fori_loop(0, ROWS, body, 0)

        @pl.when(sid == NSUB - 1)
        def _tail():
            pltpu.sync_copy(s_h.at[pl.ds(c * L, L)], s_v.at[pl.ds(0, L)])
            pltpu.sync_copy(e_h.at[pl.ds(c * L, L)], e_v.at[pl.ds(0, L)])
            pltpu.sync_copy(m_h.at[pl.ds(c * L, L)], m_v.at[pl.ds(0, L)])
            s_v[pl.ds(L, 16)] = neg16
            e_v[pl.ds(L, 16)] = jnp.zeros((16,), jnp.float32)
            m_v[pl.ds(L, 16)] = jnp.zeros((16,), jnp.float32)
            pltpu.sync_copy(si_h, si_v)
            pltpu.sync_copy(ei_h, ei_v)

            def body(kk, _):
                sl = pl.ds(kk * 16, 16)
                iv = si_v[sl]
                jv = ei_v[sl]
                sv = plsc.load_gather(s_v, [iv])
                ev = plsc.load_gather(e_v, [jv])
                mi = plsc.load_gather(m_v, [iv])
                mj = plsc.load_gather(m_v, [jv])
                lg_v[sl] = sv + ev - 1e7 * (1.0 - mi * mj)
                return 0

            lax.fori_loop(0, CHUNK // 16, body, 0)

        def body_max(kk, vmax):
            return jnp.maximum(vmax, lg_v[pl.ds(kk * 16, 16)])

        vmax = lax.fori_loop(0, CHUNK // 16, body_max, neg16)

        st_v[...] = vmax
        pltpu.sync_copy(st_v, shr_max.at[pl.ds(sid * 16, 16)])
        plsc.subcore_barrier()
        pltpu.sync_copy(shr_max, rd_v)
        gv = neg16
        for i in range(NSUB):
            gv = jnp.maximum(gv, rd_v[pl.ds(i * 16, 16)])
        gmax = lax.broadcast(jnp.max(gv), (16,))

        def body_exp(kk, vsum):
            sl = pl.ds(kk * 16, 16)
            pv = jnp.exp(lg_v[sl] - gmax)
            lg_v[sl] = pv
            return vsum + pv

        vsum = lax.fori_loop(0, CHUNK // 16, body_exp,
                             jnp.zeros((16,), jnp.float32))

        st_v[...] = vsum
        pltpu.sync_copy(st_v, shr_sum.at[pl.ds(sid * 16, 16)])
        plsc.subcore_barrier()
        pltpu.sync_copy(shr_sum, rd_v)
        tv = jnp.zeros((16,), jnp.float32)
        for i in range(NSUB):
            tv = tv + rd_v[pl.ds(i * 16, 16)]
        den = lax.broadcast(jnp.sum(tv), (16,))
        inv = 1.0 / den
        inv = inv * (2.0 - den * inv)
        inv = inv * (2.0 - den * inv)

        def body_norm(kk, carry):
            sl = pl.ds(kk * 16, 16)
            lg_v[sl] = lg_v[sl] * inv
            return carry

        lax.fori_loop(0, CHUNK // 16, body_norm, 0)

        pltpu.sync_copy(lg_v, out_h.at[c, pl.ds(base, CHUNK)])

    return k(s, e, m, si_tail, ei_tail)


def kernel(embeddings, token_type_ids, attention_mask, W, b):
    w2 = jnp.zeros((D, 2), jnp.float32)
    w2 = w2.at[: D // 2, 0].set(W[:, 0]).at[D // 2 :, 1].set(W[:, 0])
    b2 = b.reshape(1, 1)

    s, e, m = _matvec(embeddings, token_type_ids, attention_mask, w2, b2)
    out = _sc_band_softmax(s.reshape(-1), e.reshape(-1), m.reshape(-1),
                           _SI_TAIL, _EI_TAIL)
    return out[:, :P]


# speedup vs baseline: 41.1386x; 1.1512x over previous
import functools

import numpy as np
import jax
import jax.numpy as jnp
from jax import lax
from jax.experimental import pallas as pl
from jax.experimental.pallas import tpu as pltpu
from jax.experimental.pallas import tpu_sc as plsc

L = 2048
A = 16
B = 2
D = 2048
P = L * A - A * (A - 1) // 2
PP = 32768
LP = L + 16
NSUB = 16
CHUNK = PP // NSUB
ROWS = L // NSUB
TAIL = P - (NSUB - 1) * CHUNK
NEG = -1e30


def _build_pair_indices():
    i = np.arange(L).reshape(-1, 1)
    j = np.arange(L).reshape(1, -1)
    cond = (j >= i) & (j <= i + A - 1)
    idx = np.argwhere(cond)
    si = np.full((CHUNK,), L, dtype=np.int32)
    ei = np.full((CHUNK,), L, dtype=np.int32)
    base = (NSUB - 1) * CHUNK
    si[:TAIL] = idx[base:, 0]
    ei[:TAIL] = idx[base:, 1]
    return jnp.asarray(si), jnp.asarray(ei)


_SI_TAIL, _EI_TAIL = _build_pair_indices()


BLK = 512


def _matvec_body(emb_ref, tt_ref, am_ref, w_ref, b_ref, s_ref, e_ref, m_ref):
    bi = pl.program_id(0)
    li = pl.program_id(1)
    blk = emb_ref[0]
    w = w_ref[...]
    z = jnp.zeros((D // 2, 1), jnp.float32)
    w2 = jnp.concatenate(
        [jnp.concatenate([w, z], axis=1), jnp.concatenate([z, w], axis=1)],
        axis=0,
    )
    se = jnp.dot(blk, w2, preferred_element_type=jnp.float32)
    bias = b_ref[0, 0]
    sl = pl.ds(li * BLK, BLK)
    fl = pl.ds(bi * L + li * BLK, BLK)
    s_ref[fl] = se[:, 0] + bias
    e_ref[fl] = se[:, 1]
    m_ref[fl] = (tt_ref[bi, sl] * am_ref[bi, sl]).astype(jnp.float32)


def _matvec(emb, tt, am, W, b2):
    full = pl.BlockSpec((B, L), lambda i, l: (0, 0))
    flat = pl.BlockSpec((B * L,), lambda i, l: (0,))
    s, e, m = pl.pallas_call(
        _matvec_body,
        grid=(B, L // BLK),
        in_specs=[
            pl.BlockSpec((1, BLK, D), lambda i, l: (i, l, 0)),
            full,
            full,
            pl.BlockSpec((D // 2, 1), lambda i, l: (0, 0)),
            pl.BlockSpec((1, 1), lambda i, l: (0, 0)),
        ],
        out_specs=[flat, flat, flat],
        out_shape=[
            jax.ShapeDtypeStruct((B * L,), jnp.float32),
            jax.ShapeDtypeStruct((B * L,), jnp.float32),
            jax.ShapeDtypeStruct((B * L,), jnp.float32),
        ],
    )(emb, tt, am, W, b2)
    return s, e, m


def _sc_band_softmax(s, e, m, si_tail, ei_tail):
    mesh = plsc.VectorSubcoreMesh(core_axis_name="c", subcore_axis_name="s")

    @functools.partial(
        pl.kernel,
        mesh=mesh,
        out_type=jax.ShapeDtypeStruct((B, PP), jnp.float32),
        compiler_params=pltpu.CompilerParams(needs_layout_passes=False),
        scratch_types=[
            pltpu.VMEM((LP,), jnp.float32),
            pltpu.VMEM((LP,), jnp.float32),
            pltpu.VMEM((LP,), jnp.float32),
            pltpu.VMEM((CHUNK,), jnp.int32),
            pltpu.VMEM((CHUNK,), jnp.int32),
            pltpu.VMEM((ROWS + 128,), jnp.float32),
            pltpu.VMEM((ROWS + 128,), jnp.float32),
            pltpu.VMEM((ROWS + 128,), jnp.float32),
            pltpu.VMEM((CHUNK,), jnp.float32),
            pltpu.VMEM((16,), jnp.float32),
            pltpu.VMEM((NSUB * 16,), jnp.float32),
            pltpu.VMEM_SHARED((NSUB * 16,), jnp.float32),
            pltpu.VMEM_SHARED((NSUB * 16,), jnp.float32),
        ],
    )
    def k(s_h, e_h, m_h, si_h, ei_h, out_h,
          s_v, e_v, m_v, si_v, ei_v, e_lv, m_lv, s_lv,
          lg_v, st_v, rd_v, shr_max, shr_sum):
        c = lax.axis_index("c")
        sid = lax.axis_index("s")
        base = sid * CHUNK
        row0 = sid * ROWS
        neg16 = jnp.full((16,), NEG, jnp.float32)

        hoff = c * L + row0

        @pl.when(sid < NSUB - 1)
        def _regular():
            pltpu.sync_copy(e_h.at[pl.ds(hoff, ROWS + 128)], e_lv)
            pltpu.sync_copy(m_h.at[pl.ds(hoff, ROWS + 128)], m_lv)
            pltpu.sync_copy(s_h.at[pl.ds(hoff, ROWS + 128)], s_lv)

            def body(kk, _):
                kv = lax.broadcast(kk, (16,))
                sv = plsc.load_gather(s_lv, [kv])
                miv = plsc.load_gather(m_lv, [kv])
                ev = e_lv[pl.ds(kk, 16)]
                mj = m_lv[pl.ds(kk, 16)]
                lg = sv + ev - 1e7 * (1.0 - miv * mj)
                lg_v[pl.ds(kk * 16, 16)] = lg
                return 0

            lax.fori_loop(0, ROWS, body, 0)

        @pl.when(sid == NSUB - 1)
        def _tail():
            pltpu.sync_copy(s_h.at[pl.ds(c * L, L)], s_v.at[pl.ds(0, L)])
            pltpu.sync_copy(e_h.at[pl.ds(c * L, L)], e_v.at[pl.ds(0, L)])
            pltpu.sync_copy(m_h.at[pl.ds(c * L, L)], m_v.at[pl.ds(0, L)])
            s_v[pl.ds(L, 16)] = neg16
            e_v[pl.ds(L, 16)] = jnp.zeros((16,), jnp.float32)
            m_v[pl.ds(L, 16)] = jnp.zeros((16,), jnp.float32)
            pltpu.sync_copy(si_h, si_v)
            pltpu.sync_copy(ei_h, ei_v)

            def body(kk, _):
                sl = pl.ds(kk * 16, 16)
                iv = si_v[sl]
                jv = ei_v[sl]
                sv = plsc.load_gather(s_v, [iv])
                ev = plsc.load_gather(e_v, [jv])
                mi = plsc.load_gather(m_v, [iv])
                mj = plsc.load_gather(m_v, [jv])
                lg_v[sl] = sv + ev - 1e7 * (1.0 - mi * mj)
                return 0

            lax.fori_loop(0, CHUNK // 16, body, 0)

        def body_max(kk, vmax):
            return jnp.maximum(vmax, lg_v[pl.ds(kk * 16, 16)])

        vmax = lax.fori_loop(0, CHUNK // 16, body_max, neg16)

        st_v[...] = vmax
        pltpu.sync_copy(st_v, shr_max.at[pl.ds(sid * 16, 16)])
        plsc.subcore_barrier()
        pltpu.sync_copy(shr_max, rd_v)
        gv = neg16
        for i in range(NSUB):
            gv = jnp.maximum(gv, rd_v[pl.ds(i * 16, 16)])
        gmax = lax.broadcast(jnp.max(gv), (16,))

        def body_exp(kk, vsum):
            sl = pl.ds(kk * 16, 16)
            pv = jnp.exp(lg_v[sl] - gmax)
            lg_v[sl] = pv
            return vsum + pv

        vsum = lax.fori_loop(0, CHUNK // 16, body_exp,
                             jnp.zeros((16,), jnp.float32))

        st_v[...] = vsum
        pltpu.sync_copy(st_v, shr_sum.at[pl.ds(sid * 16, 16)])
        plsc.subcore_barrier()
        pltpu.sync_copy(shr_sum, rd_v)
        tv = jnp.zeros((16,), jnp.float32)
        for i in range(NSUB):
            tv = tv + rd_v[pl.ds(i * 16, 16)]
        den = lax.broadcast(jnp.sum(tv), (16,))
        inv = 1.0 / den
        inv = inv * (2.0 - den * inv)
        inv = inv * (2.0 - den * inv)

        def body_norm(kk, carry):
            sl = pl.ds(kk * 16, 16)
            lg_v[sl] = lg_v[sl] * inv
            return carry

        lax.fori_loop(0, CHUNK // 16, body_norm, 0)

        pltpu.sync_copy(lg_v, out_h.at[c, pl.ds(base, CHUNK)])

    return k(s, e, m, si_tail, ei_tail)


def kernel(embeddings, token_type_ids, attention_mask, W, b):
    s, e, m = _matvec(embeddings, token_type_ids, attention_mask,
                      W, b.reshape(1, 1))
    return _sc_band_softmax(s, e, m, _SI_TAIL, _EI_TAIL)[:, :P]


# speedup vs baseline: 42.0312x; 1.0217x over previous
import functools

import numpy as np
import jax
import jax.numpy as jnp
from jax import lax
from jax.experimental import pallas as pl
from jax.experimental.pallas import tpu as pltpu
from jax.experimental.pallas import tpu_sc as plsc

L = 2048
A = 16
B = 2
D = 2048
P = L * A - A * (A - 1) // 2
PP = 32768
LP = L + 16
NSUB = 16
CHUNK = PP // NSUB
ROWS = L // NSUB
TAIL = P - (NSUB - 1) * CHUNK
NEG = -1e30


ROW0_TAIL = (NSUB - 1) * ROWS
REG_TAIL = L - A + 1 - ROW0_TAIL
NGATH = ROWS - REG_TAIL


def _build_pair_indices():
    i = np.arange(L).reshape(-1, 1)
    j = np.arange(L).reshape(1, -1)
    cond = (j >= i) & (j <= i + A - 1)
    idx = np.argwhere(cond)
    si = np.full((NGATH * 16,), ROWS, dtype=np.int32)
    ei = np.full((NGATH * 16,), ROWS, dtype=np.int32)
    base = (NSUB - 1) * CHUNK + REG_TAIL * 16
    si[: P - base] = idx[base:, 0] - ROW0_TAIL
    ei[: P - base] = idx[base:, 1] - ROW0_TAIL
    return jnp.asarray(si), jnp.asarray(ei)


_SI_TAIL, _EI_TAIL = _build_pair_indices()


BLK = 512


def _matvec_body(emb_ref, tt_ref, am_ref, w_ref, b_ref, s_ref, e_ref, m_ref):
    bi = pl.program_id(0)
    li = pl.program_id(1)
    blk = emb_ref[0]
    w = w_ref[...]
    z = jnp.zeros((D // 2, 1), jnp.float32)
    w2 = jnp.concatenate(
        [jnp.concatenate([w, z], axis=1), jnp.concatenate([z, w], axis=1)],
        axis=0,
    )
    se = jnp.dot(blk, w2, preferred_element_type=jnp.float32)
    bias = b_ref[0, 0]
    sl = pl.ds(li * BLK, BLK)
    fl = pl.ds(bi * L + li * BLK, BLK)
    s_ref[fl] = se[:, 0] + bias
    e_ref[fl] = se[:, 1]
    m_ref[fl] = (tt_ref[bi, sl] * am_ref[bi, sl]).astype(jnp.float32)


def _matvec(emb, tt, am, W, b2):
    full = pl.BlockSpec((B, L), lambda i, l: (0, 0))
    flat = pl.BlockSpec((B * L,), lambda i, l: (0,))
    s, e, m = pl.pallas_call(
        _matvec_body,
        grid=(B, L // BLK),
        in_specs=[
            pl.BlockSpec((1, BLK, D), lambda i, l: (i, l, 0)),
            full,
            full,
            pl.BlockSpec((D // 2, 1), lambda i, l: (0, 0)),
            pl.BlockSpec((1, 1), lambda i, l: (0, 0)),
        ],
        out_specs=[flat, flat, flat],
        out_shape=[
            jax.ShapeDtypeStruct((B * L,), jnp.float32),
            jax.ShapeDtypeStruct((B * L,), jnp.float32),
            jax.ShapeDtypeStruct((B * L,), jnp.float32),
        ],
    )(emb, tt, am, W, b2)
    return s, e, m


def _sc_band_softmax(s, e, m, si_tail, ei_tail):
    mesh = plsc.VectorSubcoreMesh(core_axis_name="c", subcore_axis_name="s")

    @functools.partial(
        pl.kernel,
        mesh=mesh,
        out_type=jax.ShapeDtypeStruct((B, PP), jnp.float32),
        compiler_params=pltpu.CompilerParams(needs_layout_passes=False),
        scratch_types=[
            pltpu.VMEM((ROWS + 128,), jnp.float32),
            pltpu.VMEM((ROWS + 128,), jnp.float32),
            pltpu.VMEM((ROWS + 128,), jnp.float32),
            pltpu.VMEM((NGATH * 16,), jnp.int32),
            pltpu.VMEM((NGATH * 16,), jnp.int32),
            pltpu.VMEM((CHUNK,), jnp.float32),
            pltpu.VMEM((16,), jnp.float32),
            pltpu.VMEM((NSUB * 16,), jnp.float32),
            pltpu.VMEM_SHARED((NSUB * 16,), jnp.float32),
            pltpu.VMEM_SHARED((NSUB * 16,), jnp.float32),
        ],
    )
    def k(s_h, e_h, m_h, si_h, ei_h, out_h,
          e_lv, m_lv, s_lv, si_v, ei_v,
          lg_v, st_v, rd_v, shr_max, shr_sum):
        c = lax.axis_index("c")
        sid = lax.axis_index("s")
        base = sid * CHUNK
        row0 = sid * ROWS
        neg16 = jnp.full((16,), NEG, jnp.float32)
        hoff = c * L + row0

        def structured(kk, _):
            kv = lax.broadcast(kk, (16,))
            sv = plsc.load_gather(s_lv, [kv])
            miv = plsc.load_gather(m_lv, [kv])
            ev = e_lv[pl.ds(kk, 16)]
            mj = m_lv[pl.ds(kk, 16)]
            lg = sv + ev - 1e7 * (1.0 - miv * mj)
            lg_v[pl.ds(kk * 16, 16)] = lg
            return 0

        @pl.when(sid < NSUB - 1)
        def _regular():
            pltpu.sync_copy(e_h.at[pl.ds(hoff, ROWS + 128)], e_lv)
            pltpu.sync_copy(m_h.at[pl.ds(hoff, ROWS + 128)], m_lv)
            pltpu.sync_copy(s_h.at[pl.ds(hoff, ROWS + 128)], s_lv)
            lax.fori_loop(0, ROWS, structured, 0)

        @pl.when(sid == NSUB - 1)
        def _tail():
            pltpu.sync_copy(e_h.at[pl.ds(hoff, ROWS)], e_lv.at[pl.ds(0, ROWS)])
            pltpu.sync_copy(m_h.at[pl.ds(hoff, ROWS)], m_lv.at[pl.ds(0, ROWS)])
            pltpu.sync_copy(s_h.at[pl.ds(hoff, ROWS)], s_lv.at[pl.ds(0, ROWS)])
            s_lv[pl.ds(ROWS, 16)] = neg16
            e_lv[pl.ds(ROWS, 16)] = jnp.zeros((16,), jnp.float32)
            m_lv[pl.ds(ROWS, 16)] = jnp.zeros((16,), jnp.float32)
            pltpu.sync_copy(si_h, si_v)
            pltpu.sync_copy(ei_h, ei_v)
            lax.fori_loop(0, REG_TAIL, structured, 0)

            def gath(kk2, _):
                sl = pl.ds(kk2 * 16, 16)
                iv = si_v[sl]
                jv = ei_v[sl]
                sv = plsc.load_gather(s_lv, [iv])
                ev = plsc.load_gather(e_lv, [jv])
                mi = plsc.load_gather(m_lv, [iv])
                mj = plsc.load_gather(m_lv, [jv])
                lg_v[pl.ds((REG_TAIL + kk2) * 16, 16)] = (
                    sv + ev - 1e7 * (1.0 - mi * mj))
                return 0

            lax.fori_loop(0, NGATH, gath, 0)

        def body_max(kk, vmax):
            return jnp.maximum(vmax, lg_v[pl.ds(kk * 16, 16)])

        vmax = lax.fori_loop(0, CHUNK // 16, body_max, neg16)

        st_v[...] = vmax
        pltpu.sync_copy(st_v, shr_max.at[pl.ds(sid * 16, 16)])
        plsc.subcore_barrier()
        pltpu.sync_copy(shr_max, rd_v)
        gv = neg16
        for i in range(NSUB):
            gv = jnp.maximum(gv, rd_v[pl.ds(i * 16, 16)])
        gmax = lax.broadcast(jnp.max(gv), (16,))

        def body_exp(kk, vsum):
            sl = pl.ds(kk * 16, 16)
            pv = jnp.exp(lg_v[sl] - gmax)
            lg_v[sl] = pv
            return vsum + pv

        vsum = lax.fori_loop(0, CHUNK // 16, body_exp,
                             jnp.zeros((16,), jnp.float32))

        st_v[...] = vsum
        pltpu.sync_copy(st_v, shr_sum.at[pl.ds(sid * 16, 16)])
        plsc.subcore_barrier()
        pltpu.sync_copy(shr_sum, rd_v)
        tv = jnp.zeros((16,), jnp.float32)
        for i in range(NSUB):
            tv = tv + rd_v[pl.ds(i * 16, 16)]
        den = lax.broadcast(jnp.sum(tv), (16,))
        inv = 1.0 / den
        inv = inv * (2.0 - den * inv)
        inv = inv * (2.0 - den * inv)

        def body_norm(kk, carry):
            sl = pl.ds(kk * 16, 16)
            lg_v[sl] = lg_v[sl] * inv
            return carry

        lax.fori_loop(0, CHUNK // 16, body_norm, 0)

        pltpu.sync_copy(lg_v, out_h.at[c, pl.ds(base, CHUNK)])

    return k(s, e, m, si_tail, ei_tail)


def kernel(embeddings, token_type_ids, attention_mask, W, b):
    s, e, m = _matvec(embeddings, token_type_ids, attention_mask,
                      W, b.reshape(1, 1))
    return _sc_band_softmax(s, e, m, _SI_TAIL, _EI_TAIL)[:, :P]
